# Initial kernel scaffold; baseline (speedup 1.0000x reference)
import jax
import jax.numpy as jnp
from jax.experimental import pallas as pl


def kernel(intent, state, W1, b1, W2, b2, ln_w, ln_b, Wg, bg):
    raise NotImplementedError("write your pallas kernel here")


# speedup vs baseline: 1.1224x; 1.1224x over previous
import functools

import jax
import jax.numpy as jnp
from jax.experimental import pallas as pl
from jax.experimental.pallas import tpu as pltpu

D = 2048
N_EXPERTS = 64
TOP_K = 8
TILE = 128


def _fusion_kernel(x_ref, w1_ref, b1_ref,
                   w2_ref, b2_ref, lnw_ref, lnb_ref, wg_ref, bg_ref,
                   weights_ref, ti_ref, probs_ref, fused_ref):
    h = jax.lax.dot_general(
        x_ref[...], w1_ref[...], (((1,), (0,)), ((), ())),
        precision=jax.lax.Precision.DEFAULT,
        preferred_element_type=jnp.float32)
    h += b1_ref[...]
    h = 0.5 * h * (1.0 + jax.lax.erf(h * 0.7071067811865476))
    h2 = jax.lax.dot_general(
        h, w2_ref[...], (((1,), (0,)), ((), ())),
        precision=jax.lax.Precision.DEFAULT,
        preferred_element_type=jnp.float32)
    h2 += b2_ref[...]

    mu = jnp.mean(h2, axis=-1, keepdims=True)
    c = h2 - mu
    var = jnp.mean(c * c, axis=-1, keepdims=True)
    fused = c / jnp.sqrt(var + 1e-5) * lnw_ref[...] + lnb_ref[...]
    fused_ref[...] = fused

    logits = jax.lax.dot_general(
        fused, wg_ref[...], (((1,), (0,)), ((), ())),
        precision=jax.lax.Precision.DEFAULT,
        preferred_element_type=jnp.float32)
    logits += bg_ref[...]

    probs_ref[...] = jax.nn.softmax(logits, axis=-1)

    iota = jax.lax.broadcasted_iota(jnp.int32, logits.shape, 1)
    cur = logits
    vals = []
    idxs = []
    for _ in range(TOP_K):
        m = jnp.max(cur, axis=-1, keepdims=True)
        idx = jnp.min(jnp.where(cur == m, iota, N_EXPERTS), axis=-1,
                      keepdims=True)
        vals.append(m)
        idxs.append(idx)
        cur = jnp.where(iota == idx, -jnp.inf, cur)
    tv = jnp.concatenate(vals, axis=-1)
    ti_ref[...] = jnp.concatenate(idxs, axis=-1)
    weights_ref[...] = jax.nn.softmax(tv, axis=-1)


@jax.jit
def kernel(intent, state, W1, b1, W2, b2, ln_w, ln_b, Wg, bg):
    tokens = intent.shape[0]
    x = jnp.concatenate([intent, state], axis=-1)
    w1t = W1.T
    w2t = W2.T
    wgt = Wg.T
    b1r = b1.reshape(1, D)
    b2r = b2.reshape(1, D)
    lnwr = ln_w.reshape(1, D)
    lnbr = ln_b.reshape(1, D)
    bgr = bg.reshape(1, N_EXPERTS)

    grid = (tokens // TILE,)
    x_spec = pl.BlockSpec((TILE, 2 * D), lambda i: (i, 0))
    const = lambda shape: pl.BlockSpec(shape, lambda i: (0, 0))

    out_shapes = (
        jax.ShapeDtypeStruct((tokens, TOP_K), jnp.float32),
        jax.ShapeDtypeStruct((tokens, TOP_K), jnp.int32),
        jax.ShapeDtypeStruct((tokens, N_EXPERTS), jnp.float32),
        jax.ShapeDtypeStruct((tokens, D), jnp.float32),
    )
    out_specs = (
        pl.BlockSpec((TILE, TOP_K), lambda i: (i, 0)),
        pl.BlockSpec((TILE, TOP_K), lambda i: (i, 0)),
        pl.BlockSpec((TILE, N_EXPERTS), lambda i: (i, 0)),
        pl.BlockSpec((TILE, D), lambda i: (i, 0)),
    )

    weights, ti, all_probs, fused = pl.pallas_call(
        _fusion_kernel,
        grid=grid,
        in_specs=[
            x_spec,
            const((2 * D, D)), const((1, D)),
            const((D, D)), const((1, D)),
            const((1, D)), const((1, D)),
            const((D, N_EXPERTS)), const((1, N_EXPERTS)),
        ],
        out_specs=out_specs,
        out_shape=out_shapes,
        compiler_params=pltpu.CompilerParams(
            dimension_semantics=("arbitrary",),
            vmem_limit_bytes=100 * 1024 * 1024,
        ),
    )(x, w1t, b1r, w2t, b2r, lnwr, lnbr, wgt, bgr)
    return weights, ti, all_probs, fused
